# baseline (speedup 1.0000x reference)
import jax
import jax.numpy as jnp
from jax import lax
from jax.experimental import pallas as pl
from jax.experimental.pallas import tpu as pltpu
from jax.experimental.pallas import tpu_sc as plsc

NC = 2
NS = 16
NW = NC * NS
CHUNK = 64
BR = 1000
L = 16


def _i0(*_):
    return jnp.int32(0)


def _elu(x):
    return jnp.where(x > 0, x, jnp.exp(x) - 1.0)


def _make_sc_aggregate(n_pad, d, k):
    rpt = n_pad // NS
    mesh = plsc.VectorSubcoreMesh(
        core_axis_name="c", subcore_axis_name="s", num_cores=NC, num_subcores=NS
    )

    kh = k // 2

    def body(row_hbm, col_hbm, wh_hbm, zacc_hbm, acc_out,
             rix, cix, gbuf0, gbuf1, sem0, sem1, acc_sh):
        c = lax.axis_index("c")
        s = lax.axis_index("s")
        wid = c * NS + s
        pltpu.sync_copy(zacc_hbm, acc_sh.at[pl.ds(s * rpt, rpt)])
        plsc.subcore_barrier()

        for ph in range(2):
            pltpu.sync_copy(row_hbm.at[wid, pl.ds(ph * kh, kh)], rix)
            pltpu.sync_copy(col_hbm.at[wid, pl.ds(ph * kh, kh)], cix)
            pltpu.async_copy(wh_hbm.at[cix.at[jnp.int32(0)]], gbuf0, sem0)
            pltpu.async_copy(wh_hbm.at[cix.at[jnp.int32(1)]], gbuf1, sem1)

            def pair_step(jj, j0):
                for b, (gb, sm) in enumerate(((gbuf0, sem0), (gbuf1, sem1))):
                    j = j0 + jnp.int32(b)
                    pltpu.make_async_copy(wh_hbm.at[cix.at[j]], gb, sm).wait()
                    pltpu.sync_copy(gb, acc_sh.at[rix.at[j]], add=True)

                    @pl.when(j + jnp.int32(2) < jnp.int32(kh))
                    def _():
                        pltpu.async_copy(
                            wh_hbm.at[cix.at[j + jnp.int32(2)]], gb, sm)
                return j0 + jnp.int32(2)

            lax.fori_loop(0, kh // 2, pair_step, jnp.int32(0))

        plsc.subcore_barrier()
        pltpu.sync_copy(acc_sh.at[pl.ds(s * rpt, rpt)],
                        acc_out.at[c, pl.ds(s * rpt, rpt)])

    return pl.kernel(
        body,
        out_type=[jax.ShapeDtypeStruct((NC, n_pad, d), jnp.float32)],
        mesh=mesh,
        scratch_types=[
            pltpu.VMEM((k // 2, CHUNK), jnp.int32),
            pltpu.VMEM((k // 2, CHUNK), jnp.int32),
            pltpu.VMEM((CHUNK, d), jnp.float32),
            pltpu.VMEM((CHUNK, d), jnp.float32),
            pltpu.SemaphoreType.DMA,
            pltpu.SemaphoreType.DMA,
            pltpu.VMEM_SHARED((n_pad, d), jnp.float32),
        ],
    )


def _make_sc_degree(n_pad, k):
    half = n_pad // 2
    mesh = plsc.VectorSubcoreMesh(
        core_axis_name="c", subcore_axis_name="s", num_cores=NC, num_subcores=NS
    )

    def body(row_hbm, zhist_hbm, deg_out, rix, hist):
        c = lax.axis_index("c")
        s = lax.axis_index("s")
        wid = c * NS + s
        pltpu.sync_copy(row_hbm.at[wid], rix)
        lanes = lax.iota(jnp.int32, L)
        ones16 = jnp.ones((L,), jnp.float32)
        for p in range(2):
            lo = jnp.int32(p * half)
            hi = jnp.int32((p + 1) * half)
            pltpu.sync_copy(zhist_hbm, hist)

            def hist_step(j, carry):
                for g in range(CHUNK // L):
                    idx = rix[j, pl.ds(g * L, L)]
                    msk = (idx >= lo) & (idx < hi)
                    flat = (idx - lo) * L + lanes
                    plsc.addupdate_scatter(hist, [flat], ones16, mask=msk)
                return carry

            lax.fori_loop(0, k, hist_step, jnp.int32(0))
            pltpu.sync_copy(
                hist, deg_out.at[wid, pl.ds(p * half * L, half * L)])

    return pl.kernel(
        body,
        out_type=[jax.ShapeDtypeStruct((NW, n_pad * L), jnp.float32)],
        mesh=mesh,
        scratch_types=[
            pltpu.VMEM((k, CHUNK), jnp.int32),
            pltpu.VMEM((half * L,), jnp.float32),
        ],
        compiler_params=pltpu.CompilerParams(needs_layout_passes=False),
    )


def _mm_body(x_ref, w_ref, o_ref):
    o_ref[...] = jnp.dot(x_ref[...], w_ref[...],
                         preferred_element_type=jnp.float32)


def _matmul(xx, w):
    n, d = xx.shape
    return pl.pallas_call(
        _mm_body,
        grid=(n // BR,),
        in_specs=[pl.BlockSpec((BR, d), lambda i: (i, _i0())),
                  pl.BlockSpec((d, d), lambda i: (_i0(), _i0()))],
        out_specs=pl.BlockSpec((BR, d), lambda i: (i, _i0())),
        out_shape=jax.ShapeDtypeStruct((n, d), jnp.float32),
    )(xx, w)


def _deg_of(deg_ref):
    dsum = jnp.sum(deg_ref[...], axis=2, keepdims=True)
    return jnp.sum(dsum, axis=0) + 1.0


def _combine0_body(acc_ref, wh_ref, deg_ref, h_ref, s1_ref, s2_ref):
    deg = _deg_of(deg_ref)
    sval = (acc_ref[0] + acc_ref[1] + wh_ref[...]) / deg
    h = _elu(sval)
    h_ref[...] = h

    @pl.when(pl.program_id(0) == 0)
    def _():
        s1_ref[...] = jnp.zeros_like(s1_ref)
        s2_ref[...] = jnp.zeros_like(s2_ref)

    s1_ref[...] += jnp.sum(h, axis=0, keepdims=True)
    s2_ref[...] += jnp.sum(h * h, axis=0, keepdims=True)


def _combine0(acc, wh, deg, n, d):
    return pl.pallas_call(
        _combine0_body,
        grid=(n // BR,),
        in_specs=[pl.BlockSpec((NC, BR, d), lambda i: (_i0(), i, _i0())),
                  pl.BlockSpec((BR, d), lambda i: (i, _i0())),
                  pl.BlockSpec((NW, BR, L), lambda i: (_i0(), i, _i0()))],
        out_specs=[pl.BlockSpec((BR, d), lambda i: (i, _i0())),
                   pl.BlockSpec((1, d), lambda i: (_i0(), _i0())),
                   pl.BlockSpec((1, d), lambda i: (_i0(), _i0()))],
        out_shape=[jax.ShapeDtypeStruct((n, d), jnp.float32),
                   jax.ShapeDtypeStruct((1, d), jnp.float32),
                   jax.ShapeDtypeStruct((1, d), jnp.float32)],
    )(acc, wh, deg)


def _make_bn_mm_body(n):
    def body(h_ref, s1_ref, s2_ref, g_ref, b_ref, w_ref, o_ref):
        mean = s1_ref[...] / n
        var = s2_ref[...] / n - mean * mean
        scale = g_ref[...] * lax.rsqrt(var + 1e-5)
        hn = jnp.maximum((h_ref[...] - mean) * scale + b_ref[...], 0.0)
        o_ref[...] = jnp.dot(hn, w_ref[...],
                             preferred_element_type=jnp.float32)
    return body


def _bn_mm(h, s1, s2, g, b, w):
    n, d = h.shape
    return pl.pallas_call(
        _make_bn_mm_body(float(n)),
        grid=(n // BR,),
        in_specs=[pl.BlockSpec((BR, d), lambda i: (i, _i0())),
                  pl.BlockSpec((1, d), lambda i: (_i0(), _i0())),
                  pl.BlockSpec((1, d), lambda i: (_i0(), _i0())),
                  pl.BlockSpec((1, d), lambda i: (_i0(), _i0())),
                  pl.BlockSpec((1, d), lambda i: (_i0(), _i0())),
                  pl.BlockSpec((d, d), lambda i: (_i0(), _i0()))],
        out_specs=pl.BlockSpec((BR, d), lambda i: (i, _i0())),
        out_shape=jax.ShapeDtypeStruct((n, d), jnp.float32),
    )(h, s1, s2, g, b, w)


def _combine1_body(acc_ref, wh_ref, deg_ref, o_ref):
    deg = _deg_of(deg_ref)
    o_ref[...] = _elu((acc_ref[0] + acc_ref[1] + wh_ref[...]) / deg)


def _combine1(acc, wh, deg, n, d):
    return pl.pallas_call(
        _combine1_body,
        grid=(n // BR,),
        in_specs=[pl.BlockSpec((NC, BR, d), lambda i: (_i0(), i, _i0())),
                  pl.BlockSpec((BR, d), lambda i: (i, _i0())),
                  pl.BlockSpec((NW, BR, L), lambda i: (_i0(), i, _i0()))],
        out_specs=pl.BlockSpec((BR, d), lambda i: (i, _i0())),
        out_shape=jax.ShapeDtypeStruct((n, d), jnp.float32),
    )(acc, wh, deg)


def kernel(x, edge_index, W0, W1, bn0_gamma, bn0_beta):
    n, d = x.shape
    e = edge_index.shape[1]
    row = edge_index[0].astype(jnp.int32)
    col = edge_index[1].astype(jnp.int32)

    k = -(-e // (NW * CHUNK))
    k += (-k) % 4
    pad = NW * k * CHUNK - e
    row3 = jnp.concatenate([row, jnp.full((pad,), n, jnp.int32)]
                           ).reshape(NW, k, CHUNK)
    col3 = jnp.concatenate([col, jnp.zeros((pad,), jnp.int32)]
                           ).reshape(NW, k, CHUNK)

    gran = NS * 8
    n_pad = ((n + 1 + gran - 1) // gran) * gran
    rpt = n_pad // NS
    zacc = jnp.zeros((rpt, d), jnp.float32)
    zhist = jnp.zeros((n_pad // 2 * L,), jnp.float32)

    sc_agg = _make_sc_aggregate(n_pad, d, k)
    sc_deg = _make_sc_degree(n_pad, k)

    (deg,) = sc_deg(row3, zhist)
    deg = deg.reshape(NW, n_pad, L)
    wh0 = _matmul(x.astype(jnp.float32), W0)
    (acc0,) = sc_agg(row3, col3, wh0, zacc)
    h, s1, s2 = _combine0(acc0, wh0, deg, n, d)
    wh1 = _bn_mm(h, s1, s2, bn0_gamma.reshape(1, d), bn0_beta.reshape(1, d), W1)
    (acc1,) = sc_agg(row3, col3, wh1, zacc)
    return _combine1(acc1, wh1, deg, n, d)


# speedup vs baseline: 1.0229x; 1.0229x over previous
import jax
import jax.numpy as jnp
from jax import lax
from jax.experimental import pallas as pl
from jax.experimental.pallas import tpu as pltpu
from jax.experimental.pallas import tpu_sc as plsc

NC = 2
NS = 16
NW = NC * NS
CHUNK = 128
BR = 1000
L = 16


def _i0(*_):
    return jnp.int32(0)


def _elu(x):
    return jnp.where(x > 0, x, jnp.exp(x) - 1.0)


def _make_sc_aggregate(n_pad, d, k):
    rpt = n_pad // NS
    mesh = plsc.VectorSubcoreMesh(
        core_axis_name="c", subcore_axis_name="s", num_cores=NC, num_subcores=NS
    )

    kh = k // 2

    def body(row_hbm, col_hbm, wh_hbm, zacc_hbm, acc_out,
             rix, cix, gbuf0, gbuf1, sem0, sem1, acc_sh):
        c = lax.axis_index("c")
        s = lax.axis_index("s")
        wid = c * NS + s
        pltpu.sync_copy(zacc_hbm, acc_sh.at[pl.ds(s * rpt, rpt)])
        plsc.subcore_barrier()

        for ph in range(2):
            pltpu.sync_copy(row_hbm.at[wid, pl.ds(ph * kh, kh)], rix)
            pltpu.sync_copy(col_hbm.at[wid, pl.ds(ph * kh, kh)], cix)
            pltpu.async_copy(wh_hbm.at[cix.at[jnp.int32(0)]], gbuf0, sem0)
            pltpu.async_copy(wh_hbm.at[cix.at[jnp.int32(1)]], gbuf1, sem1)

            def pair_step(jj, j0):
                for b, (gb, sm) in enumerate(((gbuf0, sem0), (gbuf1, sem1))):
                    j = j0 + jnp.int32(b)
                    pltpu.make_async_copy(wh_hbm.at[cix.at[j]], gb, sm).wait()
                    pltpu.sync_copy(gb, acc_sh.at[rix.at[j]], add=True)

                    @pl.when(j + jnp.int32(2) < jnp.int32(kh))
                    def _():
                        pltpu.async_copy(
                            wh_hbm.at[cix.at[j + jnp.int32(2)]], gb, sm)
                return j0 + jnp.int32(2)

            lax.fori_loop(0, kh // 2, pair_step, jnp.int32(0))

        plsc.subcore_barrier()
        pltpu.sync_copy(acc_sh.at[pl.ds(s * rpt, rpt)],
                        acc_out.at[c, pl.ds(s * rpt, rpt)])

    return pl.kernel(
        body,
        out_type=[jax.ShapeDtypeStruct((NC, n_pad, d), jnp.float32)],
        mesh=mesh,
        scratch_types=[
            pltpu.VMEM((k // 2, CHUNK), jnp.int32),
            pltpu.VMEM((k // 2, CHUNK), jnp.int32),
            pltpu.VMEM((CHUNK, d), jnp.float32),
            pltpu.VMEM((CHUNK, d), jnp.float32),
            pltpu.SemaphoreType.DMA,
            pltpu.SemaphoreType.DMA,
            pltpu.VMEM_SHARED((n_pad, d), jnp.float32),
        ],
    )


def _make_sc_degree(n_pad, k):
    half = n_pad // 2
    mesh = plsc.VectorSubcoreMesh(
        core_axis_name="c", subcore_axis_name="s", num_cores=NC, num_subcores=NS
    )

    def body(row_hbm, zhist_hbm, deg_out, rix, hist):
        c = lax.axis_index("c")
        s = lax.axis_index("s")
        wid = c * NS + s
        pltpu.sync_copy(row_hbm.at[wid], rix)
        lanes = lax.iota(jnp.int32, L)
        ones16 = jnp.ones((L,), jnp.float32)
        for p in range(2):
            lo = jnp.int32(p * half)
            hi = jnp.int32((p + 1) * half)
            pltpu.sync_copy(zhist_hbm, hist)

            def hist_step(j, carry):
                for g in range(CHUNK // L):
                    idx = rix[j, pl.ds(g * L, L)]
                    msk = (idx >= lo) & (idx < hi)
                    flat = (idx - lo) * L + lanes
                    plsc.addupdate_scatter(hist, [flat], ones16, mask=msk)
                return carry

            lax.fori_loop(0, k, hist_step, jnp.int32(0))
            pltpu.sync_copy(
                hist, deg_out.at[wid, pl.ds(p * half * L, half * L)])

    return pl.kernel(
        body,
        out_type=[jax.ShapeDtypeStruct((NW, n_pad * L), jnp.float32)],
        mesh=mesh,
        scratch_types=[
            pltpu.VMEM((k, CHUNK), jnp.int32),
            pltpu.VMEM((half * L,), jnp.float32),
        ],
        compiler_params=pltpu.CompilerParams(needs_layout_passes=False),
    )


def _mm_body(x_ref, w_ref, o_ref):
    o_ref[...] = jnp.dot(x_ref[...], w_ref[...],
                         preferred_element_type=jnp.float32)


def _matmul(xx, w):
    n, d = xx.shape
    return pl.pallas_call(
        _mm_body,
        grid=(n // BR,),
        in_specs=[pl.BlockSpec((BR, d), lambda i: (i, _i0())),
                  pl.BlockSpec((d, d), lambda i: (_i0(), _i0()))],
        out_specs=pl.BlockSpec((BR, d), lambda i: (i, _i0())),
        out_shape=jax.ShapeDtypeStruct((n, d), jnp.float32),
    )(xx, w)


def _deg_of(deg_ref):
    dsum = jnp.sum(deg_ref[...], axis=2, keepdims=True)
    return jnp.sum(dsum, axis=0) + 1.0


def _combine0_body(acc_ref, wh_ref, deg_ref, h_ref, s1_ref, s2_ref):
    deg = _deg_of(deg_ref)
    sval = (acc_ref[0] + acc_ref[1] + wh_ref[...]) / deg
    h = _elu(sval)
    h_ref[...] = h

    @pl.when(pl.program_id(0) == 0)
    def _():
        s1_ref[...] = jnp.zeros_like(s1_ref)
        s2_ref[...] = jnp.zeros_like(s2_ref)

    s1_ref[...] += jnp.sum(h, axis=0, keepdims=True)
    s2_ref[...] += jnp.sum(h * h, axis=0, keepdims=True)


def _combine0(acc, wh, deg, n, d):
    return pl.pallas_call(
        _combine0_body,
        grid=(n // BR,),
        in_specs=[pl.BlockSpec((NC, BR, d), lambda i: (_i0(), i, _i0())),
                  pl.BlockSpec((BR, d), lambda i: (i, _i0())),
                  pl.BlockSpec((NW, BR, L), lambda i: (_i0(), i, _i0()))],
        out_specs=[pl.BlockSpec((BR, d), lambda i: (i, _i0())),
                   pl.BlockSpec((1, d), lambda i: (_i0(), _i0())),
                   pl.BlockSpec((1, d), lambda i: (_i0(), _i0()))],
        out_shape=[jax.ShapeDtypeStruct((n, d), jnp.float32),
                   jax.ShapeDtypeStruct((1, d), jnp.float32),
                   jax.ShapeDtypeStruct((1, d), jnp.float32)],
    )(acc, wh, deg)


def _make_bn_mm_body(n):
    def body(h_ref, s1_ref, s2_ref, g_ref, b_ref, w_ref, o_ref):
        mean = s1_ref[...] / n
        var = s2_ref[...] / n - mean * mean
        scale = g_ref[...] * lax.rsqrt(var + 1e-5)
        hn = jnp.maximum((h_ref[...] - mean) * scale + b_ref[...], 0.0)
        o_ref[...] = jnp.dot(hn, w_ref[...],
                             preferred_element_type=jnp.float32)
    return body


def _bn_mm(h, s1, s2, g, b, w):
    n, d = h.shape
    return pl.pallas_call(
        _make_bn_mm_body(float(n)),
        grid=(n // BR,),
        in_specs=[pl.BlockSpec((BR, d), lambda i: (i, _i0())),
                  pl.BlockSpec((1, d), lambda i: (_i0(), _i0())),
                  pl.BlockSpec((1, d), lambda i: (_i0(), _i0())),
                  pl.BlockSpec((1, d), lambda i: (_i0(), _i0())),
                  pl.BlockSpec((1, d), lambda i: (_i0(), _i0())),
                  pl.BlockSpec((d, d), lambda i: (_i0(), _i0()))],
        out_specs=pl.BlockSpec((BR, d), lambda i: (i, _i0())),
        out_shape=jax.ShapeDtypeStruct((n, d), jnp.float32),
    )(h, s1, s2, g, b, w)


def _combine1_body(acc_ref, wh_ref, deg_ref, o_ref):
    deg = _deg_of(deg_ref)
    o_ref[...] = _elu((acc_ref[0] + acc_ref[1] + wh_ref[...]) / deg)


def _combine1(acc, wh, deg, n, d):
    return pl.pallas_call(
        _combine1_body,
        grid=(n // BR,),
        in_specs=[pl.BlockSpec((NC, BR, d), lambda i: (_i0(), i, _i0())),
                  pl.BlockSpec((BR, d), lambda i: (i, _i0())),
                  pl.BlockSpec((NW, BR, L), lambda i: (_i0(), i, _i0()))],
        out_specs=pl.BlockSpec((BR, d), lambda i: (i, _i0())),
        out_shape=jax.ShapeDtypeStruct((n, d), jnp.float32),
    )(acc, wh, deg)


def kernel(x, edge_index, W0, W1, bn0_gamma, bn0_beta):
    n, d = x.shape
    e = edge_index.shape[1]
    row = edge_index[0].astype(jnp.int32)
    col = edge_index[1].astype(jnp.int32)

    k = -(-e // (NW * CHUNK))
    k += (-k) % 4
    pad = NW * k * CHUNK - e
    row3 = jnp.concatenate([row, jnp.full((pad,), n, jnp.int32)]
                           ).reshape(NW, k, CHUNK)
    col3 = jnp.concatenate([col, jnp.zeros((pad,), jnp.int32)]
                           ).reshape(NW, k, CHUNK)

    gran = NS * 8
    n_pad = ((n + 1 + gran - 1) // gran) * gran
    rpt = n_pad // NS
    zacc = jnp.zeros((rpt, d), jnp.float32)
    zhist = jnp.zeros((n_pad // 2 * L,), jnp.float32)

    sc_agg = _make_sc_aggregate(n_pad, d, k)
    sc_deg = _make_sc_degree(n_pad, k)

    (deg,) = sc_deg(row3, zhist)
    deg = deg.reshape(NW, n_pad, L)
    wh0 = _matmul(x.astype(jnp.float32), W0)
    (acc0,) = sc_agg(row3, col3, wh0, zacc)
    h, s1, s2 = _combine0(acc0, wh0, deg, n, d)
    wh1 = _bn_mm(h, s1, s2, bn0_gamma.reshape(1, d), bn0_beta.reshape(1, d), W1)
    (acc1,) = sc_agg(row3, col3, wh1, zacc)
    return _combine1(acc1, wh1, deg, n, d)


# speedup vs baseline: 1.4863x; 1.4530x over previous
import jax
import jax.numpy as jnp
from jax import lax
from jax.experimental import pallas as pl
from jax.experimental.pallas import tpu as pltpu
from jax.experimental.pallas import tpu_sc as plsc

NC = 2
NS = 16
NW = NC * NS
CHUNK = 128
BR = 1000
L = 16


def _i0(*_):
    return jnp.int32(0)


def _elu(x):
    return jnp.where(x > 0, x, jnp.exp(x) - 1.0)


def _make_sc_aggregate(n_pad, d, k):
    rpt = n_pad // NS
    mesh = plsc.VectorSubcoreMesh(
        core_axis_name="c", subcore_axis_name="s", num_cores=NC, num_subcores=NS
    )

    def body(row_hbm, col_hbm, wh_hbm, zacc_hbm, acc_out, rix, cix, gbuf, acc_sh):
        c = lax.axis_index("c")
        s = lax.axis_index("s")
        wid = c * NS + s
        pltpu.sync_copy(row_hbm.at[wid], rix)
        pltpu.sync_copy(col_hbm.at[wid], cix)
        pltpu.sync_copy(zacc_hbm, acc_sh.at[pl.ds(s * rpt, rpt)])
        plsc.subcore_barrier()

        def chunk_step(j, carry):
            pltpu.sync_copy(wh_hbm.at[cix.at[j]], gbuf)
            pltpu.sync_copy(gbuf, acc_sh.at[rix.at[j]], add=True)
            return carry

        lax.fori_loop(0, k, chunk_step, jnp.int32(0))
        plsc.subcore_barrier()
        pltpu.sync_copy(acc_sh.at[pl.ds(s * rpt, rpt)],
                        acc_out.at[c, pl.ds(s * rpt, rpt)])

    return pl.kernel(
        body,
        out_type=[jax.ShapeDtypeStruct((NC, n_pad, d), jnp.float32)],
        mesh=mesh,
        scratch_types=[
            pltpu.VMEM((k, CHUNK), jnp.int32),
            pltpu.VMEM((k, CHUNK), jnp.int32),
            pltpu.VMEM((CHUNK, d), jnp.float32),
            pltpu.VMEM_SHARED((n_pad, d), jnp.float32),
        ],
    )


def _mm_body(x_ref, w_ref, o_ref):
    o_ref[...] = jnp.dot(x_ref[...], w_ref[...],
                         preferred_element_type=jnp.float32)


def _matmul(xx, w):
    n, d = xx.shape
    return pl.pallas_call(
        _mm_body,
        grid=(n // BR,),
        in_specs=[pl.BlockSpec((BR, d), lambda i: (i, _i0())),
                  pl.BlockSpec((d, d), lambda i: (_i0(), _i0()))],
        out_specs=pl.BlockSpec((BR, d), lambda i: (i, _i0())),
        out_shape=jax.ShapeDtypeStruct((n, d), jnp.float32),
    )(xx, w)


EB = 8192


def _hist_body(r_ref, c_ref):
    r = r_ref[...]
    hi = lax.div(r, jnp.int32(128))
    lo = r - hi * jnp.int32(128)
    io = lax.broadcasted_iota(jnp.int32, (EB, 128), 1)
    mhi = (hi == io).astype(jnp.float32)
    mlo = (lo == io).astype(jnp.float32)

    @pl.when(pl.program_id(0) == 0)
    def _():
        c_ref[...] = jnp.zeros_like(c_ref)

    c_ref[...] += lax.dot_general(mhi, mlo, (((0,), (0,)), ((), ())),
                                  preferred_element_type=jnp.float32)


def _tc_degree(row_h):
    eh = row_h.shape[0]
    return pl.pallas_call(
        _hist_body,
        grid=(eh // EB,),
        in_specs=[pl.BlockSpec((EB, 1), lambda i: (i, _i0()))],
        out_specs=pl.BlockSpec((128, 128), lambda i: (_i0(), _i0())),
        out_shape=jax.ShapeDtypeStruct((128, 128), jnp.float32),
    )(row_h)


def _deg_of(deg_ref):
    return deg_ref[...] + 1.0


def _combine0_body(acc_ref, wh_ref, deg_ref, h_ref, s1_ref, s2_ref):
    deg = _deg_of(deg_ref)
    sval = (acc_ref[0] + acc_ref[1] + wh_ref[...]) / deg
    h = _elu(sval)
    h_ref[...] = h

    @pl.when(pl.program_id(0) == 0)
    def _():
        s1_ref[...] = jnp.zeros_like(s1_ref)
        s2_ref[...] = jnp.zeros_like(s2_ref)

    s1_ref[...] += jnp.sum(h, axis=0, keepdims=True)
    s2_ref[...] += jnp.sum(h * h, axis=0, keepdims=True)


def _combine0(acc, wh, deg, n, d):
    return pl.pallas_call(
        _combine0_body,
        grid=(n // BR,),
        in_specs=[pl.BlockSpec((NC, BR, d), lambda i: (_i0(), i, _i0())),
                  pl.BlockSpec((BR, d), lambda i: (i, _i0())),
                  pl.BlockSpec((BR, 1), lambda i: (i, _i0()))],
        out_specs=[pl.BlockSpec((BR, d), lambda i: (i, _i0())),
                   pl.BlockSpec((1, d), lambda i: (_i0(), _i0())),
                   pl.BlockSpec((1, d), lambda i: (_i0(), _i0()))],
        out_shape=[jax.ShapeDtypeStruct((n, d), jnp.float32),
                   jax.ShapeDtypeStruct((1, d), jnp.float32),
                   jax.ShapeDtypeStruct((1, d), jnp.float32)],
    )(acc, wh, deg)


def _make_bn_mm_body(n):
    def body(h_ref, s1_ref, s2_ref, g_ref, b_ref, w_ref, o_ref):
        mean = s1_ref[...] / n
        var = s2_ref[...] / n - mean * mean
        scale = g_ref[...] * lax.rsqrt(var + 1e-5)
        hn = jnp.maximum((h_ref[...] - mean) * scale + b_ref[...], 0.0)
        o_ref[...] = jnp.dot(hn, w_ref[...],
                             preferred_element_type=jnp.float32)
    return body


def _bn_mm(h, s1, s2, g, b, w):
    n, d = h.shape
    return pl.pallas_call(
        _make_bn_mm_body(float(n)),
        grid=(n // BR,),
        in_specs=[pl.BlockSpec((BR, d), lambda i: (i, _i0())),
                  pl.BlockSpec((1, d), lambda i: (_i0(), _i0())),
                  pl.BlockSpec((1, d), lambda i: (_i0(), _i0())),
                  pl.BlockSpec((1, d), lambda i: (_i0(), _i0())),
                  pl.BlockSpec((1, d), lambda i: (_i0(), _i0())),
                  pl.BlockSpec((d, d), lambda i: (_i0(), _i0()))],
        out_specs=pl.BlockSpec((BR, d), lambda i: (i, _i0())),
        out_shape=jax.ShapeDtypeStruct((n, d), jnp.float32),
    )(h, s1, s2, g, b, w)


def _combine1_body(acc_ref, wh_ref, deg_ref, o_ref):
    deg = _deg_of(deg_ref)
    o_ref[...] = _elu((acc_ref[0] + acc_ref[1] + wh_ref[...]) / deg)


def _combine1(acc, wh, deg, n, d):
    return pl.pallas_call(
        _combine1_body,
        grid=(n // BR,),
        in_specs=[pl.BlockSpec((NC, BR, d), lambda i: (_i0(), i, _i0())),
                  pl.BlockSpec((BR, d), lambda i: (i, _i0())),
                  pl.BlockSpec((BR, 1), lambda i: (i, _i0()))],
        out_specs=pl.BlockSpec((BR, d), lambda i: (i, _i0())),
        out_shape=jax.ShapeDtypeStruct((n, d), jnp.float32),
    )(acc, wh, deg)


def kernel(x, edge_index, W0, W1, bn0_gamma, bn0_beta):
    n, d = x.shape
    e = edge_index.shape[1]
    row = edge_index[0].astype(jnp.int32)
    col = edge_index[1].astype(jnp.int32)

    k = -(-e // (NW * CHUNK))
    pad = NW * k * CHUNK - e
    row3 = jnp.concatenate([row, jnp.full((pad,), n, jnp.int32)]
                           ).reshape(NW, k, CHUNK)
    col3 = jnp.concatenate([col, jnp.zeros((pad,), jnp.int32)]
                           ).reshape(NW, k, CHUNK)

    gran = NS * 8
    n_pad = ((n + 1 + gran - 1) // gran) * gran
    rpt = n_pad // NS
    zacc = jnp.zeros((rpt, d), jnp.float32)

    sc_agg = _make_sc_aggregate(n_pad, d, k)

    eh = -(-e // EB) * EB
    row_h = jnp.concatenate([row, jnp.full((eh - e,), n, jnp.int32)]
                            ).reshape(eh, 1)
    wh0 = _matmul(x.astype(jnp.float32), W0)
    (acc0,) = sc_agg(row3, col3, wh0, zacc)
    deg = _tc_degree(row_h).reshape(128 * 128, 1)
    h, s1, s2 = _combine0(acc0, wh0, deg, n, d)
    wh1 = _bn_mm(h, s1, s2, bn0_gamma.reshape(1, d), bn0_beta.reshape(1, d), W1)
    (acc1,) = sc_agg(row3, col3, wh1, zacc)
    return _combine1(acc1, wh1, deg, n, d)


# speedup vs baseline: 2.4649x; 1.6584x over previous
import jax
import jax.numpy as jnp
from jax import lax
from jax.experimental import pallas as pl
from jax.experimental.pallas import tpu as pltpu
from jax.experimental.pallas import tpu_sc as plsc

NC = 2
NS = 16
NW = NC * NS
CHUNK = 128
BR = 1000
L = 16


def _i0(*_):
    return jnp.int32(0)


def _elu(x):
    return jnp.where(x > 0, x, jnp.exp(x) - 1.0)


def _make_sc_aggregate(n_pad, d, k0, k1):
    rpt = n_pad // NS
    kmax = max(k0, k1)
    mesh = plsc.VectorSubcoreMesh(
        core_axis_name="c", subcore_axis_name="s", num_cores=NC, num_subcores=NS
    )

    def body(row_hbm, col_hbm, wh_hbm, zacc_hbm, acc_out, rix, cix, gbuf, acc_sh):
        c = lax.axis_index("c")
        s = lax.axis_index("s")
        wid = c * NS + s
        kc = jnp.where(c == jnp.int32(0), jnp.int32(k0), jnp.int32(k1))
        pltpu.sync_copy(row_hbm.at[wid], rix)
        pltpu.sync_copy(col_hbm.at[wid], cix)
        pltpu.sync_copy(zacc_hbm, acc_sh.at[pl.ds(s * rpt, rpt)])
        plsc.subcore_barrier()

        def chunk_step(j, carry):
            pltpu.sync_copy(wh_hbm.at[cix.at[j]], gbuf)
            pltpu.sync_copy(gbuf, acc_sh.at[rix.at[j]], add=True)
            return carry

        lax.fori_loop(jnp.int32(0), kc, chunk_step, jnp.int32(0))
        plsc.subcore_barrier()
        pltpu.sync_copy(acc_sh.at[pl.ds(s * rpt, rpt)],
                        acc_out.at[c, pl.ds(s * rpt, rpt)])

    return pl.kernel(
        body,
        out_type=[jax.ShapeDtypeStruct((NC, n_pad, d), jnp.float32)],
        mesh=mesh,
        scratch_types=[
            pltpu.VMEM((kmax, CHUNK), jnp.int32),
            pltpu.VMEM((kmax, CHUNK), jnp.int32),
            pltpu.VMEM((CHUNK, d), jnp.float32),
            pltpu.VMEM_SHARED((n_pad, d), jnp.float32),
        ],
    )


def _mm_body(x_ref, w_ref, o_ref):
    o_ref[...] = jnp.dot(x_ref[...], w_ref[...],
                         preferred_element_type=jnp.float32)


def _matmul(xx, w):
    n, d = xx.shape
    return pl.pallas_call(
        _mm_body,
        grid=(n // BR,),
        in_specs=[pl.BlockSpec((BR, d), lambda i: (i, _i0())),
                  pl.BlockSpec((d, d), lambda i: (_i0(), _i0()))],
        out_specs=pl.BlockSpec((BR, d), lambda i: (i, _i0())),
        out_shape=jax.ShapeDtypeStruct((n, d), jnp.float32),
    )(xx, w)


EB = 8192


def _hist_body(r_ref, c_ref):
    r = r_ref[...]
    hi = lax.div(r, jnp.int32(128))
    lo = r - hi * jnp.int32(128)
    io = lax.broadcasted_iota(jnp.int32, (128, EB), 0)
    mhi = (hi == io).astype(jnp.float32)
    mlo = (lo == io).astype(jnp.float32)

    @pl.when(pl.program_id(0) == 0)
    def _():
        c_ref[...] = jnp.zeros_like(c_ref)

    c_ref[...] += lax.dot_general(mhi, mlo, (((1,), (1,)), ((), ())),
                                  preferred_element_type=jnp.float32)


def _tc_degree(row_h):
    eh = row_h.shape[1]
    return pl.pallas_call(
        _hist_body,
        grid=(eh // EB,),
        in_specs=[pl.BlockSpec((1, EB), lambda i: (_i0(), i))],
        out_specs=pl.BlockSpec((128, 128), lambda i: (_i0(), _i0())),
        out_shape=jax.ShapeDtypeStruct((128, 128), jnp.float32),
    )(row_h)


def _deg_of(deg_ref):
    return deg_ref[...] + 1.0


def _combine0_body(acc_ref, wh_ref, deg_ref, h_ref, s1_ref, s2_ref):
    deg = _deg_of(deg_ref)
    sval = (acc_ref[0] + acc_ref[1] + wh_ref[...]) / deg
    h = _elu(sval)
    h_ref[...] = h

    @pl.when(pl.program_id(0) == 0)
    def _():
        s1_ref[...] = jnp.zeros_like(s1_ref)
        s2_ref[...] = jnp.zeros_like(s2_ref)

    s1_ref[...] += jnp.sum(h, axis=0, keepdims=True)
    s2_ref[...] += jnp.sum(h * h, axis=0, keepdims=True)


def _combine0(acc, wh, deg, n, d):
    return pl.pallas_call(
        _combine0_body,
        grid=(n // BR,),
        in_specs=[pl.BlockSpec((NC, BR, d), lambda i: (_i0(), i, _i0())),
                  pl.BlockSpec((BR, d), lambda i: (i, _i0())),
                  pl.BlockSpec((BR, 1), lambda i: (i, _i0()))],
        out_specs=[pl.BlockSpec((BR, d), lambda i: (i, _i0())),
                   pl.BlockSpec((1, d), lambda i: (_i0(), _i0())),
                   pl.BlockSpec((1, d), lambda i: (_i0(), _i0()))],
        out_shape=[jax.ShapeDtypeStruct((n, d), jnp.float32),
                   jax.ShapeDtypeStruct((1, d), jnp.float32),
                   jax.ShapeDtypeStruct((1, d), jnp.float32)],
    )(acc, wh, deg)


def _make_bn_mm_body(n):
    def body(h_ref, s1_ref, s2_ref, g_ref, b_ref, w_ref, o_ref):
        mean = s1_ref[...] / n
        var = s2_ref[...] / n - mean * mean
        scale = g_ref[...] * lax.rsqrt(var + 1e-5)
        hn = jnp.maximum((h_ref[...] - mean) * scale + b_ref[...], 0.0)
        o_ref[...] = jnp.dot(hn, w_ref[...],
                             preferred_element_type=jnp.float32)
    return body


def _bn_mm(h, s1, s2, g, b, w):
    n, d = h.shape
    return pl.pallas_call(
        _make_bn_mm_body(float(n)),
        grid=(n // BR,),
        in_specs=[pl.BlockSpec((BR, d), lambda i: (i, _i0())),
                  pl.BlockSpec((1, d), lambda i: (_i0(), _i0())),
                  pl.BlockSpec((1, d), lambda i: (_i0(), _i0())),
                  pl.BlockSpec((1, d), lambda i: (_i0(), _i0())),
                  pl.BlockSpec((1, d), lambda i: (_i0(), _i0())),
                  pl.BlockSpec((d, d), lambda i: (_i0(), _i0()))],
        out_specs=pl.BlockSpec((BR, d), lambda i: (i, _i0())),
        out_shape=jax.ShapeDtypeStruct((n, d), jnp.float32),
    )(h, s1, s2, g, b, w)


def _combine1_body(acc_ref, wh_ref, deg_ref, o_ref):
    deg = _deg_of(deg_ref)
    o_ref[...] = _elu((acc_ref[0] + acc_ref[1] + wh_ref[...]) / deg)


def _combine1(acc, wh, deg, n, d):
    return pl.pallas_call(
        _combine1_body,
        grid=(n // BR,),
        in_specs=[pl.BlockSpec((NC, BR, d), lambda i: (_i0(), i, _i0())),
                  pl.BlockSpec((BR, d), lambda i: (i, _i0())),
                  pl.BlockSpec((BR, 1), lambda i: (i, _i0()))],
        out_specs=pl.BlockSpec((BR, d), lambda i: (i, _i0())),
        out_shape=jax.ShapeDtypeStruct((n, d), jnp.float32),
    )(acc, wh, deg)


def kernel(x, edge_index, W0, W1, bn0_gamma, bn0_beta):
    n, d = x.shape
    e = edge_index.shape[1]
    row = edge_index[0].astype(jnp.int32)
    col = edge_index[1].astype(jnp.int32)

    R0 = 0.59
    tot = -(-e // CHUNK)
    k0 = max(1, min(tot // NS, round(R0 * tot / NS)))
    k1 = max(1, -(-(tot - NS * k0) // NS))
    kmax = max(k0, k1)
    cap0 = NS * k0 * CHUNK
    cap1 = NS * k1 * CHUNK
    padn = cap0 + cap1 - e
    rowp = jnp.concatenate([row, jnp.full((padn,), n, jnp.int32)])
    colp = jnp.concatenate([col, jnp.zeros((padn,), jnp.int32)])

    def _split(a, fill):
        a0 = a[:cap0].reshape(NS, k0, CHUNK)
        a1 = a[cap0:].reshape(NS, k1, CHUNK)
        a0 = jnp.pad(a0, ((0, 0), (0, kmax - k0), (0, 0)), constant_values=fill)
        a1 = jnp.pad(a1, ((0, 0), (0, kmax - k1), (0, 0)), constant_values=fill)
        return jnp.concatenate([a0, a1], axis=0)

    row3 = _split(rowp, n)
    col3 = _split(colp, 0)

    gran = NS * 8
    n_pad = ((n + 1 + gran - 1) // gran) * gran
    rpt = n_pad // NS
    zacc = jnp.zeros((rpt, d), jnp.float32)

    sc_agg = _make_sc_aggregate(n_pad, d, k0, k1)

    eh = -(-e // EB) * EB
    row_h = jnp.concatenate([row, jnp.full((eh - e,), n, jnp.int32)]
                            ).reshape(1, eh)
    wh0 = _matmul(x.astype(jnp.float32), W0)
    (acc0,) = sc_agg(row3, col3, wh0, zacc)
    deg = _tc_degree(row_h).reshape(128 * 128, 1)
    h, s1, s2 = _combine0(acc0, wh0, deg, n, d)
    wh1 = _bn_mm(h, s1, s2, bn0_gamma.reshape(1, d), bn0_beta.reshape(1, d), W1)
    (acc1,) = sc_agg(row3, col3, wh1, zacc)
    return _combine1(acc1, wh1, deg, n, d)
